# initial kernel scaffold (unmeasured)
import jax
import jax.numpy as jnp
from jax import lax
from jax.experimental import pallas as pl
from jax.experimental.pallas import tpu as pltpu

T_HALF = 2048
T_FULL = 4096
D = 1024
F = 2048
E_LOCAL = 4
CHUNK = 512


def _pairwise_barrier(peer):
    barrier_sem = pltpu.get_barrier_semaphore()
    pl.semaphore_signal(
        barrier_sem, inc=1, device_id=peer,
        device_id_type=pl.DeviceIdType.MESH,
    )
    pl.semaphore_wait(barrier_sem, 1)


def _exchange_body(x_ref, a_ref, xf_ref, af_ref, send_sems, recv_sems):
    my_x = lax.axis_index("x")
    my_y = lax.axis_index("y")
    my_z = lax.axis_index("z")
    peer = (1 - my_x, my_y, my_z)

    _pairwise_barrier(peer)

    rows = pl.ds(my_x * T_HALF, T_HALF)
    xf_ref[rows, :] = x_ref[:, :]
    af_ref[rows, :] = a_ref[:, :]

    rd_x = pltpu.make_async_remote_copy(
        src_ref=x_ref,
        dst_ref=xf_ref.at[rows, :],
        send_sem=send_sems.at[0],
        recv_sem=recv_sems.at[0],
        device_id=peer,
        device_id_type=pl.DeviceIdType.MESH,
    )
    rd_a = pltpu.make_async_remote_copy(
        src_ref=a_ref,
        dst_ref=af_ref.at[rows, :],
        send_sem=send_sems.at[1],
        recv_sem=recv_sems.at[1],
        device_id=peer,
        device_id_type=pl.DeviceIdType.MESH,
    )
    rd_x.start()
    rd_a.start()
    rd_x.wait()
    rd_a.wait()


def _moe_body(xf_ref, af_ref, w1_hbm, w2_hbm, out_ref, w1_buf, w2_buf, dma_sems):
    my_x = lax.axis_index("x")
    for e in range(E_LOCAL):
        cp1 = pltpu.make_async_copy(w1_hbm.at[e], w1_buf, dma_sems.at[0])
        cp2 = pltpu.make_async_copy(w2_hbm.at[e], w2_buf, dma_sems.at[1])
        cp1.start()
        cp2.start()
        cp1.wait()
        cp2.wait()
        e_global = E_LOCAL * my_x + e
        for c in range(T_FULL // CHUNK):
            rows = pl.ds(c * CHUNK, CHUNK)
            xb = xf_ref[rows, :]
            h = jnp.maximum(
                jnp.dot(xb, w1_buf[:, :], preferred_element_type=jnp.float32),
                0.0,
            )
            o = jnp.dot(h, w2_buf[:, :], preferred_element_type=jnp.float32)
            mask = af_ref[rows, :] == e_global
            if e == 0:
                out_ref[rows, :] = jnp.where(mask, o, 0.0)
            else:
                out_ref[rows, :] = jnp.where(mask, o, out_ref[rows, :])


def _combine_body(c_ref, out_ref, recv_buf, send_sem, recv_sem):
    my_x = lax.axis_index("x")
    my_y = lax.axis_index("y")
    my_z = lax.axis_index("z")
    peer = (1 - my_x, my_y, my_z)

    _pairwise_barrier(peer)

    peer_rows = pl.ds((1 - my_x) * T_HALF, T_HALF)
    my_rows = pl.ds(my_x * T_HALF, T_HALF)
    rd = pltpu.make_async_remote_copy(
        src_ref=c_ref.at[peer_rows, :],
        dst_ref=recv_buf,
        send_sem=send_sem,
        recv_sem=recv_sem,
        device_id=peer,
        device_id_type=pl.DeviceIdType.MESH,
    )
    rd.start()
    rd.wait()
    out_ref[:, :] = c_ref[my_rows, :] + recv_buf[:, :]


def kernel(x, assign, W1, W2):
    a2 = assign.reshape(T_HALF, 1)

    xf, af = pl.pallas_call(
        _exchange_body,
        out_shape=(
            jax.ShapeDtypeStruct((T_FULL, D), jnp.float32),
            jax.ShapeDtypeStruct((T_FULL, 1), jnp.int32),
        ),
        in_specs=[
            pl.BlockSpec(memory_space=pltpu.VMEM),
            pl.BlockSpec(memory_space=pltpu.VMEM),
        ],
        out_specs=(
            pl.BlockSpec(memory_space=pltpu.VMEM),
            pl.BlockSpec(memory_space=pltpu.VMEM),
        ),
        scratch_shapes=[
            pltpu.SemaphoreType.DMA((2,)),
            pltpu.SemaphoreType.DMA((2,)),
        ],
        compiler_params=pltpu.CompilerParams(collective_id=0),
    )(x, a2)

    contrib = pl.pallas_call(
        _moe_body,
        out_shape=jax.ShapeDtypeStruct((T_FULL, D), jnp.float32),
        in_specs=[
            pl.BlockSpec(memory_space=pltpu.VMEM),
            pl.BlockSpec(memory_space=pltpu.VMEM),
            pl.BlockSpec(memory_space=pltpu.ANY),
            pl.BlockSpec(memory_space=pltpu.ANY),
        ],
        out_specs=pl.BlockSpec(memory_space=pltpu.VMEM),
        scratch_shapes=[
            pltpu.VMEM((D, F), jnp.float32),
            pltpu.VMEM((F, D), jnp.float32),
            pltpu.SemaphoreType.DMA((2,)),
        ],
    )(xf, af, W1, W2)

    out = pl.pallas_call(
        _combine_body,
        out_shape=jax.ShapeDtypeStruct((T_HALF, D), jnp.float32),
        in_specs=[pl.BlockSpec(memory_space=pltpu.VMEM)],
        out_specs=pl.BlockSpec(memory_space=pltpu.VMEM),
        scratch_shapes=[
            pltpu.VMEM((T_HALF, D), jnp.float32),
            pltpu.SemaphoreType.DMA,
            pltpu.SemaphoreType.DMA,
        ],
        compiler_params=pltpu.CompilerParams(collective_id=1),
    )(contrib)

    return out


# baseline (device time: 410078 ns/iter reference)
import jax
import jax.numpy as jnp
from jax import lax
from jax.experimental import pallas as pl
from jax.experimental.pallas import tpu as pltpu

T_HALF = 2048
T_FULL = 4096
D = 1024
F = 2048
E_LOCAL = 4
CHUNK = 512
VMEM_LIMIT = 63 * 1024 * 1024


def _pairwise_barrier(peer):
    barrier_sem = pltpu.get_barrier_semaphore()
    pl.semaphore_signal(
        barrier_sem, inc=1, device_id=peer,
        device_id_type=pl.DeviceIdType.MESH,
    )
    pl.semaphore_wait(barrier_sem, 1)


def _exchange_body(x_ref, a_ref, xf_ref, af_ref, send_sems, recv_sems):
    my_x = lax.axis_index("x")
    my_y = lax.axis_index("y")
    my_z = lax.axis_index("z")
    peer = (1 - my_x, my_y, my_z)

    _pairwise_barrier(peer)

    rows = pl.ds(my_x * T_HALF, T_HALF)
    xf_ref[rows, :] = x_ref[:, :]
    af_ref[rows, :] = a_ref[:, :]

    rd_x = pltpu.make_async_remote_copy(
        src_ref=x_ref,
        dst_ref=xf_ref.at[rows, :],
        send_sem=send_sems.at[0],
        recv_sem=recv_sems.at[0],
        device_id=peer,
        device_id_type=pl.DeviceIdType.MESH,
    )
    rd_a = pltpu.make_async_remote_copy(
        src_ref=a_ref,
        dst_ref=af_ref.at[rows, :],
        send_sem=send_sems.at[1],
        recv_sem=recv_sems.at[1],
        device_id=peer,
        device_id_type=pl.DeviceIdType.MESH,
    )
    rd_x.start()
    rd_a.start()
    rd_x.wait()
    rd_a.wait()


def _moe_body(xf_ref, af_ref, w1_hbm, w2_hbm, out_ref, w1_buf, w2_buf, dma_sems):
    my_x = lax.axis_index("x")
    for e in range(E_LOCAL):
        cp1 = pltpu.make_async_copy(w1_hbm.at[e], w1_buf, dma_sems.at[0])
        cp2 = pltpu.make_async_copy(w2_hbm.at[e], w2_buf, dma_sems.at[1])
        cp1.start()
        cp2.start()
        cp1.wait()
        cp2.wait()
        e_global = E_LOCAL * my_x + e
        for c in range(T_FULL // CHUNK):
            rows = pl.ds(c * CHUNK, CHUNK)
            xb = xf_ref[rows, :]
            h = jnp.maximum(
                jnp.dot(xb, w1_buf[:, :], preferred_element_type=jnp.float32),
                0.0,
            )
            o = jnp.dot(h, w2_buf[:, :], preferred_element_type=jnp.float32)
            mask = af_ref[rows, :] == e_global
            if e == 0:
                out_ref[rows, :] = jnp.where(mask, o, 0.0)
            else:
                out_ref[rows, :] = jnp.where(mask, o, out_ref[rows, :])


def _combine_body(c_ref, out_ref, recv_buf, send_sem, recv_sem):
    my_x = lax.axis_index("x")
    my_y = lax.axis_index("y")
    my_z = lax.axis_index("z")
    peer = (1 - my_x, my_y, my_z)

    _pairwise_barrier(peer)

    peer_rows = pl.ds((1 - my_x) * T_HALF, T_HALF)
    my_rows = pl.ds(my_x * T_HALF, T_HALF)
    rd = pltpu.make_async_remote_copy(
        src_ref=c_ref.at[peer_rows, :],
        dst_ref=recv_buf,
        send_sem=send_sem,
        recv_sem=recv_sem,
        device_id=peer,
        device_id_type=pl.DeviceIdType.MESH,
    )
    rd.start()
    rd.wait()
    out_ref[:, :] = c_ref[my_rows, :] + recv_buf[:, :]


def kernel(x, assign, W1, W2):
    a2 = assign.reshape(T_HALF, 1)

    xf, af = pl.pallas_call(
        _exchange_body,
        out_shape=(
            jax.ShapeDtypeStruct((T_FULL, D), jnp.float32),
            jax.ShapeDtypeStruct((T_FULL, 1), jnp.int32),
        ),
        in_specs=[
            pl.BlockSpec(memory_space=pltpu.VMEM),
            pl.BlockSpec(memory_space=pltpu.VMEM),
        ],
        out_specs=(
            pl.BlockSpec(memory_space=pltpu.VMEM),
            pl.BlockSpec(memory_space=pltpu.VMEM),
        ),
        scratch_shapes=[
            pltpu.SemaphoreType.DMA((2,)),
            pltpu.SemaphoreType.DMA((2,)),
        ],
        compiler_params=pltpu.CompilerParams(
            collective_id=0, vmem_limit_bytes=VMEM_LIMIT
        ),
    )(x, a2)

    contrib = pl.pallas_call(
        _moe_body,
        out_shape=jax.ShapeDtypeStruct((T_FULL, D), jnp.float32),
        in_specs=[
            pl.BlockSpec(memory_space=pltpu.VMEM),
            pl.BlockSpec(memory_space=pltpu.VMEM),
            pl.BlockSpec(memory_space=pl.ANY),
            pl.BlockSpec(memory_space=pl.ANY),
        ],
        out_specs=pl.BlockSpec(memory_space=pltpu.VMEM),
        scratch_shapes=[
            pltpu.VMEM((D, F), jnp.float32),
            pltpu.VMEM((F, D), jnp.float32),
            pltpu.SemaphoreType.DMA((2,)),
        ],
        compiler_params=pltpu.CompilerParams(vmem_limit_bytes=VMEM_LIMIT),
    )(xf, af, W1, W2)

    out = pl.pallas_call(
        _combine_body,
        out_shape=jax.ShapeDtypeStruct((T_HALF, D), jnp.float32),
        in_specs=[pl.BlockSpec(memory_space=pltpu.VMEM)],
        out_specs=pl.BlockSpec(memory_space=pltpu.VMEM),
        scratch_shapes=[
            pltpu.VMEM((T_HALF, D), jnp.float32),
            pltpu.SemaphoreType.DMA,
            pltpu.SemaphoreType.DMA,
        ],
        compiler_params=pltpu.CompilerParams(
            collective_id=1, vmem_limit_bytes=VMEM_LIMIT
        ),
    )(contrib)

    return out


# device time: 293601 ns/iter; 1.3967x vs baseline; 1.3967x over previous
import jax
import jax.numpy as jnp
from jax import lax
from jax.experimental import pallas as pl
from jax.experimental.pallas import tpu as pltpu

T_HALF = 2048
T_FULL = 4096
D = 1024
F = 2048
E_LOCAL = 4
CHUNK = 512
VMEM_LIMIT = 63 * 1024 * 1024


def _pairwise_barrier(peer):
    barrier_sem = pltpu.get_barrier_semaphore()
    pl.semaphore_signal(
        barrier_sem, inc=1, device_id=peer,
        device_id_type=pl.DeviceIdType.MESH,
    )
    pl.semaphore_wait(barrier_sem, 1)


def _x_peer():
    my_x = lax.axis_index("x")
    my_y = lax.axis_index("y")
    my_z = lax.axis_index("z")
    return my_x, (1 - my_x, my_y, my_z)


def _exchange_body(x_ref, a_ref, xf_ref, af_ref, xb_ref, send_sems, recv_sems):
    my_x, peer = _x_peer()

    _pairwise_barrier(peer)

    rows = pl.ds(my_x * T_HALF, T_HALF)
    xb_ref[:, :] = x_ref[:, :].astype(jnp.bfloat16)

    rd_x = pltpu.make_async_remote_copy(
        src_ref=xb_ref,
        dst_ref=xf_ref.at[rows, :],
        send_sem=send_sems.at[0],
        recv_sem=recv_sems.at[0],
        device_id=peer,
        device_id_type=pl.DeviceIdType.MESH,
    )
    rd_a = pltpu.make_async_remote_copy(
        src_ref=a_ref,
        dst_ref=af_ref.at[rows, :],
        send_sem=send_sems.at[1],
        recv_sem=recv_sems.at[1],
        device_id=peer,
        device_id_type=pl.DeviceIdType.MESH,
    )
    rd_x.start()
    rd_a.start()

    xf_ref[rows, :] = xb_ref[:, :]
    af_ref[rows, :] = a_ref[:, :]

    rd_x.wait()
    rd_a.wait()


def _moe_body(xf_ref, af_ref, w1_hbm, w2_hbm, out_ref, w1_buf, w2_buf, dma_sems):
    my_x = lax.axis_index("x")

    def start_load(e, slot):
        cp1 = pltpu.make_async_copy(w1_hbm.at[e], w1_buf.at[slot], dma_sems.at[slot, 0])
        cp2 = pltpu.make_async_copy(w2_hbm.at[e], w2_buf.at[slot], dma_sems.at[slot, 1])
        cp1.start()
        cp2.start()
        return cp1, cp2

    pending = start_load(0, 0)
    for e in range(E_LOCAL):
        slot = e % 2
        pending[0].wait()
        pending[1].wait()
        if e + 1 < E_LOCAL:
            pending = start_load(e + 1, (e + 1) % 2)
        e_global = E_LOCAL * my_x + e

        def chunk_step(c, _, e=e, slot=slot, e_global=e_global):
            rows = pl.ds(c * CHUNK, CHUNK)
            xb = xf_ref[rows, :].astype(jnp.float32)
            h = jnp.maximum(
                jnp.dot(xb, w1_buf[slot], preferred_element_type=jnp.float32),
                0.0,
            )
            o = jnp.dot(h, w2_buf[slot], preferred_element_type=jnp.float32)
            mask = af_ref[rows, :] == e_global
            if e == 0:
                out_ref[rows, :] = jnp.where(mask, o, 0.0).astype(jnp.bfloat16)
            else:
                out_ref[rows, :] = jnp.where(
                    mask, o.astype(jnp.bfloat16), out_ref[rows, :]
                )
            return 0

        lax.fori_loop(0, T_FULL // CHUNK, chunk_step, 0)


def _combine_body(c_ref, out_ref, recv_buf, send_sem, recv_sem):
    my_x, peer = _x_peer()

    _pairwise_barrier(peer)

    peer_rows = pl.ds((1 - my_x) * T_HALF, T_HALF)
    my_rows = pl.ds(my_x * T_HALF, T_HALF)
    rd = pltpu.make_async_remote_copy(
        src_ref=c_ref.at[peer_rows, :],
        dst_ref=recv_buf,
        send_sem=send_sem,
        recv_sem=recv_sem,
        device_id=peer,
        device_id_type=pl.DeviceIdType.MESH,
    )
    rd.start()
    rd.wait()
    out_ref[:, :] = c_ref[my_rows, :].astype(jnp.float32) + recv_buf[:, :].astype(
        jnp.float32
    )


def kernel(x, assign, W1, W2):
    a2 = assign.reshape(T_HALF, 1)

    xf, af = pl.pallas_call(
        _exchange_body,
        out_shape=(
            jax.ShapeDtypeStruct((T_FULL, D), jnp.bfloat16),
            jax.ShapeDtypeStruct((T_FULL, 1), jnp.int32),
        ),
        in_specs=[
            pl.BlockSpec(memory_space=pltpu.VMEM),
            pl.BlockSpec(memory_space=pltpu.VMEM),
        ],
        out_specs=(
            pl.BlockSpec(memory_space=pltpu.VMEM),
            pl.BlockSpec(memory_space=pltpu.VMEM),
        ),
        scratch_shapes=[
            pltpu.VMEM((T_HALF, D), jnp.bfloat16),
            pltpu.SemaphoreType.DMA((2,)),
            pltpu.SemaphoreType.DMA((2,)),
        ],
        compiler_params=pltpu.CompilerParams(
            collective_id=0, vmem_limit_bytes=VMEM_LIMIT
        ),
    )(x, a2)

    contrib = pl.pallas_call(
        _moe_body,
        out_shape=jax.ShapeDtypeStruct((T_FULL, D), jnp.bfloat16),
        in_specs=[
            pl.BlockSpec(memory_space=pltpu.VMEM),
            pl.BlockSpec(memory_space=pltpu.VMEM),
            pl.BlockSpec(memory_space=pl.ANY),
            pl.BlockSpec(memory_space=pl.ANY),
        ],
        out_specs=pl.BlockSpec(memory_space=pltpu.VMEM),
        scratch_shapes=[
            pltpu.VMEM((2, D, F), jnp.float32),
            pltpu.VMEM((2, F, D), jnp.float32),
            pltpu.SemaphoreType.DMA((2, 2)),
        ],
        compiler_params=pltpu.CompilerParams(vmem_limit_bytes=VMEM_LIMIT),
    )(xf, af, W1, W2)

    out = pl.pallas_call(
        _combine_body,
        out_shape=jax.ShapeDtypeStruct((T_HALF, D), jnp.float32),
        in_specs=[pl.BlockSpec(memory_space=pltpu.VMEM)],
        out_specs=pl.BlockSpec(memory_space=pltpu.VMEM),
        scratch_shapes=[
            pltpu.VMEM((T_HALF, D), jnp.bfloat16),
            pltpu.SemaphoreType.DMA,
            pltpu.SemaphoreType.DMA,
        ],
        compiler_params=pltpu.CompilerParams(
            collective_id=1, vmem_limit_bytes=VMEM_LIMIT
        ),
    )(contrib)

    return out


# device time: 179404 ns/iter; 2.2858x vs baseline; 1.6365x over previous
import jax
import jax.numpy as jnp
from jax import lax
from jax.experimental import pallas as pl
from jax.experimental.pallas import tpu as pltpu

T_HALF = 2048
T_Q = 1024
D = 1024
F = 2048
E_LOCAL = 4
VMEM_LIMIT = 63 * 1024 * 1024


def _idx():
    return lax.axis_index("x"), lax.axis_index("y"), lax.axis_index("z")


def _pairwise_barrier(peer):
    barrier_sem = pltpu.get_barrier_semaphore()
    pl.semaphore_signal(
        barrier_sem, inc=1, device_id=peer,
        device_id_type=pl.DeviceIdType.MESH,
    )
    pl.semaphore_wait(barrier_sem, 1)


def _remote(src, dst, send_sem, recv_sem, peer):
    return pltpu.make_async_remote_copy(
        src_ref=src, dst_ref=dst, send_sem=send_sem, recv_sem=recv_sem,
        device_id=peer, device_id_type=pl.DeviceIdType.MESH,
    )


def _dispatch_body(x_ref, a_ref, xq_ref, aq_ref, send_sems, recv_sems):
    my_x, my_y, my_z = _idx()
    peer = (1 - my_x, my_y, my_z)

    _pairwise_barrier(peer)

    rd_x = _remote(xq_ref, xq_ref, send_sems.at[0], recv_sems.at[0], peer)
    rd_a = _remote(aq_ref, aq_ref, send_sems.at[1], recv_sems.at[1], peer)

    @pl.when(my_y == my_x)
    def _sender():
        rows = pl.ds(my_z * T_Q, T_Q)
        xq_ref[:, :] = x_ref[rows, :].astype(jnp.bfloat16)
        aq_ref[:, :] = a_ref[rows, :]
        rd_x.start()
        rd_a.start()
        rd_x.wait_send()
        rd_a.wait_send()

    @pl.when(my_y != my_x)
    def _receiver():
        rd_x.wait_recv()
        rd_a.wait_recv()


def _moe_body(xq_ref, aq_ref, w1_hbm, w2_hbm, part_ref, w1_buf, w2_buf, dma_sems):
    my_x = lax.axis_index("x")

    def start_load(e, slot):
        cp1 = pltpu.make_async_copy(w1_hbm.at[e], w1_buf.at[slot], dma_sems.at[slot, 0])
        cp2 = pltpu.make_async_copy(w2_hbm.at[e], w2_buf.at[slot], dma_sems.at[slot, 1])
        cp1.start()
        cp2.start()
        return cp1, cp2

    xb = xq_ref[:, :].astype(jnp.float32)
    pending = start_load(0, 0)
    for e in range(E_LOCAL):
        slot = e % 2
        pending[0].wait()
        pending[1].wait()
        if e + 1 < E_LOCAL:
            pending = start_load(e + 1, (e + 1) % 2)
        h = jnp.maximum(
            jnp.dot(xb, w1_buf[slot], preferred_element_type=jnp.float32), 0.0
        )
        o = jnp.dot(h, w2_buf[slot], preferred_element_type=jnp.float32)
        mask = aq_ref[:, :] == E_LOCAL * my_x + e
        if e == 0:
            part_ref[:, :] = jnp.where(mask, o, 0.0).astype(jnp.bfloat16)
        else:
            part_ref[:, :] = jnp.where(
                mask, o.astype(jnp.bfloat16), part_ref[:, :]
            )


def _stage_a_body(part_ref, q_ref, recv_buf, send_sem, recv_sem):
    my_x, my_y, my_z = _idx()
    peer = (1 - my_x, my_y, my_z)

    _pairwise_barrier(peer)

    rd = _remote(part_ref, recv_buf, send_sem, recv_sem, peer)
    rd.start()
    rd.wait()
    q_ref[:, :] = (
        part_ref[:, :].astype(jnp.float32) + recv_buf[:, :].astype(jnp.float32)
    ).astype(jnp.bfloat16)


def _stage_b1_body(q_ref, qh_ref, send_sems, recv_sems):
    my_x, my_y, my_z = _idx()
    peer = (my_x, 1 - my_y, my_z)

    _pairwise_barrier(peer)

    rd = _remote(q_ref, qh_ref, send_sems, recv_sems, peer)

    @pl.when(my_y == my_x)
    def _sender():
        qh_ref[:, :] = q_ref[:, :]
        rd.start()
        rd.wait_send()

    @pl.when(my_y != my_x)
    def _receiver():
        rd.wait_recv()


def _stage_b2_body(qh_ref, out_ref, recv_buf, send_sem, recv_sem):
    my_x, my_y, my_z = _idx()
    peer = (my_x, my_y, 1 - my_z)

    _pairwise_barrier(peer)

    rd = _remote(qh_ref, recv_buf, send_sem, recv_sem, peer)
    rd.start()
    out_ref[pl.ds(my_z * T_Q, T_Q), :] = qh_ref[:, :].astype(jnp.float32)
    rd.wait()
    out_ref[pl.ds((1 - my_z) * T_Q, T_Q), :] = recv_buf[:, :].astype(jnp.float32)


def kernel(x, assign, W1, W2):
    a2 = assign.reshape(T_HALF, 1)

    xq, aq = pl.pallas_call(
        _dispatch_body,
        out_shape=(
            jax.ShapeDtypeStruct((T_Q, D), jnp.bfloat16),
            jax.ShapeDtypeStruct((T_Q, 1), jnp.int32),
        ),
        in_specs=[
            pl.BlockSpec(memory_space=pltpu.VMEM),
            pl.BlockSpec(memory_space=pltpu.VMEM),
        ],
        out_specs=(
            pl.BlockSpec(memory_space=pltpu.VMEM),
            pl.BlockSpec(memory_space=pltpu.VMEM),
        ),
        scratch_shapes=[
            pltpu.SemaphoreType.DMA((2,)),
            pltpu.SemaphoreType.DMA((2,)),
        ],
        compiler_params=pltpu.CompilerParams(
            collective_id=0, vmem_limit_bytes=VMEM_LIMIT
        ),
    )(x, a2)

    part = pl.pallas_call(
        _moe_body,
        out_shape=jax.ShapeDtypeStruct((T_Q, D), jnp.bfloat16),
        in_specs=[
            pl.BlockSpec(memory_space=pltpu.VMEM),
            pl.BlockSpec(memory_space=pltpu.VMEM),
            pl.BlockSpec(memory_space=pl.ANY),
            pl.BlockSpec(memory_space=pl.ANY),
        ],
        out_specs=pl.BlockSpec(memory_space=pltpu.VMEM),
        scratch_shapes=[
            pltpu.VMEM((2, D, F), jnp.float32),
            pltpu.VMEM((2, F, D), jnp.float32),
            pltpu.SemaphoreType.DMA((2, 2)),
        ],
        compiler_params=pltpu.CompilerParams(vmem_limit_bytes=VMEM_LIMIT),
    )(xq, aq, W1, W2)

    q = pl.pallas_call(
        _stage_a_body,
        out_shape=jax.ShapeDtypeStruct((T_Q, D), jnp.bfloat16),
        in_specs=[pl.BlockSpec(memory_space=pltpu.VMEM)],
        out_specs=pl.BlockSpec(memory_space=pltpu.VMEM),
        scratch_shapes=[
            pltpu.VMEM((T_Q, D), jnp.bfloat16),
            pltpu.SemaphoreType.DMA,
            pltpu.SemaphoreType.DMA,
        ],
        compiler_params=pltpu.CompilerParams(
            collective_id=1, vmem_limit_bytes=VMEM_LIMIT
        ),
    )(part)

    qh = pl.pallas_call(
        _stage_b1_body,
        out_shape=jax.ShapeDtypeStruct((T_Q, D), jnp.bfloat16),
        in_specs=[pl.BlockSpec(memory_space=pltpu.VMEM)],
        out_specs=pl.BlockSpec(memory_space=pltpu.VMEM),
        scratch_shapes=[
            pltpu.SemaphoreType.DMA,
            pltpu.SemaphoreType.DMA,
        ],
        compiler_params=pltpu.CompilerParams(
            collective_id=2, vmem_limit_bytes=VMEM_LIMIT
        ),
    )(q)

    out = pl.pallas_call(
        _stage_b2_body,
        out_shape=jax.ShapeDtypeStruct((T_HALF, D), jnp.float32),
        in_specs=[pl.BlockSpec(memory_space=pltpu.VMEM)],
        out_specs=pl.BlockSpec(memory_space=pltpu.VMEM),
        scratch_shapes=[
            pltpu.VMEM((T_Q, D), jnp.bfloat16),
            pltpu.SemaphoreType.DMA,
            pltpu.SemaphoreType.DMA,
        ],
        compiler_params=pltpu.CompilerParams(
            collective_id=3, vmem_limit_bytes=VMEM_LIMIT
        ),
    )(qh)

    return out


# device time: 139170 ns/iter; 2.9466x vs baseline; 1.2891x over previous
import jax
import jax.numpy as jnp
from jax import lax
from jax.experimental import pallas as pl
from jax.experimental.pallas import tpu as pltpu

T_HALF = 2048
T_Q = 1024
D = 1024
F = 2048
E_LOCAL = 4
CHUNK = 512
N_CHUNK = T_Q // CHUNK
VMEM_LIMIT = 63 * 1024 * 1024


def _idx():
    return lax.axis_index("x"), lax.axis_index("y"), lax.axis_index("z")


def _barrier(peers):
    barrier_sem = pltpu.get_barrier_semaphore()
    for p in peers:
        pl.semaphore_signal(
            barrier_sem, inc=1, device_id=p,
            device_id_type=pl.DeviceIdType.MESH,
        )
    pl.semaphore_wait(barrier_sem, len(peers))


def _remote(src, dst, send_sem, recv_sem, peer):
    return pltpu.make_async_remote_copy(
        src_ref=src, dst_ref=dst, send_sem=send_sem, recv_sem=recv_sem,
        device_id=peer, device_id_type=pl.DeviceIdType.MESH,
    )


def _moe_body(
    x_ref, a_ref, w1_hbm, w2_hbm, part_ref,
    xq_ref, aq_ref, w1_buf, w2_buf, dma_sems, send_sems, recv_sems,
):
    my_x, my_y, my_z = _idx()
    peer = (1 - my_x, my_y, my_z)

    _barrier([peer])

    rd_x = _remote(xq_ref, xq_ref, send_sems.at[0], recv_sems.at[0], peer)
    rd_a = _remote(aq_ref, aq_ref, send_sems.at[1], recv_sems.at[1], peer)

    @pl.when(my_y == my_x)
    def _sender():
        rows = pl.ds(my_z * T_Q, T_Q)
        xq_ref[:, :] = x_ref[rows, :].astype(jnp.bfloat16)
        aq_ref[:, :] = a_ref[rows, :]
        rd_x.start()
        rd_a.start()

    def start_load(e, slot):
        cp1 = pltpu.make_async_copy(w1_hbm.at[e], w1_buf.at[slot], dma_sems.at[slot, 0])
        cp2 = pltpu.make_async_copy(w2_hbm.at[e], w2_buf.at[slot], dma_sems.at[slot, 1])
        cp1.start()
        cp2.start()
        return cp1, cp2

    pending = start_load(0, 0)

    @pl.when(my_y == my_x)
    def _sender_done():
        rd_x.wait_send()
        rd_a.wait_send()

    @pl.when(my_y != my_x)
    def _receiver():
        rd_x.wait_recv()
        rd_a.wait_recv()

    for e in range(E_LOCAL):
        slot = e % 2
        pending[0].wait()
        pending[1].wait()
        if e + 1 < E_LOCAL:
            pending = start_load(e + 1, (e + 1) % 2)
        e_global = E_LOCAL * my_x + e
        for c in range(N_CHUNK):
            rows = pl.ds(c * CHUNK, CHUNK)
            xb = xq_ref[rows, :].astype(jnp.float32)
            h = jnp.maximum(
                jnp.dot(xb, w1_buf[slot], preferred_element_type=jnp.float32),
                0.0,
            )
            o = jnp.dot(h, w2_buf[slot], preferred_element_type=jnp.float32)
            mask = aq_ref[rows, :] == e_global
            if e == 0:
                part_ref[rows, :] = jnp.where(mask, o, 0.0).astype(jnp.bfloat16)
            else:
                part_ref[rows, :] = jnp.where(
                    mask, o.astype(jnp.bfloat16), part_ref[rows, :]
                )


def _combine_body(
    part_ref, out_ref, recv_a, q_ref, qh_ref, recv_b2, send_sems, recv_sems,
):
    my_x, my_y, my_z = _idx()
    x_peer = (1 - my_x, my_y, my_z)
    y_peer = (my_x, 1 - my_y, my_z)
    z_peer = (my_x, my_y, 1 - my_z)
    i_hold = my_y == my_x

    _barrier([x_peer, y_peer, z_peer])

    def ds(c):
        return pl.ds(c * CHUNK, CHUNK)

    rd_a = [
        _remote(part_ref.at[ds(c), :], recv_a.at[ds(c), :],
                send_sems.at[0, c], recv_sems.at[0, c], x_peer)
        for c in range(N_CHUNK)
    ]
    rd_b1 = [
        _remote(q_ref.at[ds(c), :], qh_ref.at[ds(c), :],
                send_sems.at[1, c], recv_sems.at[1, c], y_peer)
        for c in range(N_CHUNK)
    ]
    rd_b2 = [
        _remote(qh_ref.at[ds(c), :], recv_b2.at[ds(c), :],
                send_sems.at[2, c], recv_sems.at[2, c], z_peer)
        for c in range(N_CHUNK)
    ]

    for c in range(N_CHUNK):
        rd_a[c].start()

    for c in range(N_CHUNK):
        rd_a[c].wait_recv()
        q_ref[ds(c), :] = (
            part_ref[ds(c), :].astype(jnp.float32)
            + recv_a[ds(c), :].astype(jnp.float32)
        ).astype(jnp.bfloat16)

        @pl.when(i_hold)
        def _push():
            qh_ref[ds(c), :] = q_ref[ds(c), :]
            rd_b1[c].start()
            rd_b2[c].start()

    for c in range(N_CHUNK):
        @pl.when(jnp.logical_not(i_hold))
        def _recv_fwd():
            rd_b1[c].wait_recv()
            rd_b2[c].start()
        out_ref[pl.ds(my_z * T_Q + c * CHUNK, CHUNK), :] = qh_ref[
            ds(c), :
        ].astype(jnp.float32)

    for c in range(N_CHUNK):
        rd_b2[c].wait_recv()
        out_ref[pl.ds((1 - my_z) * T_Q + c * CHUNK, CHUNK), :] = recv_b2[
            ds(c), :
        ].astype(jnp.float32)

    for c in range(N_CHUNK):
        rd_a[c].wait_send()
        rd_b2[c].wait_send()

        @pl.when(i_hold)
        def _done():
            rd_b1[c].wait_send()


def kernel(x, assign, W1, W2):
    a2 = assign.reshape(T_HALF, 1)

    part = pl.pallas_call(
        _moe_body,
        out_shape=jax.ShapeDtypeStruct((T_Q, D), jnp.bfloat16),
        in_specs=[
            pl.BlockSpec(memory_space=pltpu.VMEM),
            pl.BlockSpec(memory_space=pltpu.VMEM),
            pl.BlockSpec(memory_space=pl.ANY),
            pl.BlockSpec(memory_space=pl.ANY),
        ],
        out_specs=pl.BlockSpec(memory_space=pltpu.VMEM),
        scratch_shapes=[
            pltpu.VMEM((T_Q, D), jnp.bfloat16),
            pltpu.VMEM((T_Q, 1), jnp.int32),
            pltpu.VMEM((2, D, F), jnp.float32),
            pltpu.VMEM((2, F, D), jnp.float32),
            pltpu.SemaphoreType.DMA((2, 2)),
            pltpu.SemaphoreType.DMA((2,)),
            pltpu.SemaphoreType.DMA((2,)),
        ],
        compiler_params=pltpu.CompilerParams(
            collective_id=0, vmem_limit_bytes=VMEM_LIMIT
        ),
    )(x, a2, W1, W2)

    out = pl.pallas_call(
        _combine_body,
        out_shape=jax.ShapeDtypeStruct((T_HALF, D), jnp.float32),
        in_specs=[pl.BlockSpec(memory_space=pltpu.VMEM)],
        out_specs=pl.BlockSpec(memory_space=pltpu.VMEM),
        scratch_shapes=[
            pltpu.VMEM((T_Q, D), jnp.bfloat16),
            pltpu.VMEM((T_Q, D), jnp.bfloat16),
            pltpu.VMEM((T_Q, D), jnp.bfloat16),
            pltpu.VMEM((T_Q, D), jnp.bfloat16),
            pltpu.SemaphoreType.DMA((3, N_CHUNK)),
            pltpu.SemaphoreType.DMA((3, N_CHUNK)),
        ],
        compiler_params=pltpu.CompilerParams(
            collective_id=1, vmem_limit_bytes=VMEM_LIMIT
        ),
    )(part)

    return out


# device time: 127765 ns/iter; 3.2096x vs baseline; 1.0893x over previous
import jax
import jax.numpy as jnp
from jax import lax
from jax.experimental import pallas as pl
from jax.experimental.pallas import tpu as pltpu

T_HALF = 2048
T_Q = 1024
D = 1024
F = 2048
E_LOCAL = 4
CHUNK = 512
N_CHUNK = T_Q // CHUNK
CHUNK_C = 256
N_CHUNK_C = T_Q // CHUNK_C
VMEM_LIMIT = 63 * 1024 * 1024


def _idx():
    return lax.axis_index("x"), lax.axis_index("y"), lax.axis_index("z")


def _barrier(peers):
    barrier_sem = pltpu.get_barrier_semaphore()
    for p in peers:
        pl.semaphore_signal(
            barrier_sem, inc=1, device_id=p,
            device_id_type=pl.DeviceIdType.MESH,
        )
    pl.semaphore_wait(barrier_sem, len(peers))


def _remote(src, dst, send_sem, recv_sem, peer):
    return pltpu.make_async_remote_copy(
        src_ref=src, dst_ref=dst, send_sem=send_sem, recv_sem=recv_sem,
        device_id=peer, device_id_type=pl.DeviceIdType.MESH,
    )


def _moe_body(
    x_ref, a_ref, w1_hbm, w2_hbm, part_ref,
    xq_ref, aq_ref, w1_buf, w2_buf, dma_sems, send_sems, recv_sems,
):
    my_x, my_y, my_z = _idx()
    peer = (1 - my_x, my_y, my_z)

    _barrier([peer])

    rd_x = _remote(xq_ref, xq_ref, send_sems.at[0], recv_sems.at[0], peer)
    rd_a = _remote(aq_ref, aq_ref, send_sems.at[1], recv_sems.at[1], peer)

    @pl.when(my_y == my_x)
    def _sender():
        rows = pl.ds(my_z * T_Q, T_Q)
        xq_ref[:, :] = x_ref[rows, :].astype(jnp.bfloat16)
        aq_ref[:, :] = a_ref[rows, :]
        rd_x.start()
        rd_a.start()

    def start_load(e, slot):
        cp1 = pltpu.make_async_copy(w1_hbm.at[e], w1_buf.at[slot], dma_sems.at[slot, 0])
        cp2 = pltpu.make_async_copy(w2_hbm.at[e], w2_buf.at[slot], dma_sems.at[slot, 1])
        cp1.start()
        cp2.start()
        return cp1, cp2

    pending = start_load(0, 0)

    @pl.when(my_y == my_x)
    def _sender_done():
        rd_x.wait_send()
        rd_a.wait_send()

    @pl.when(my_y != my_x)
    def _receiver():
        rd_x.wait_recv()
        rd_a.wait_recv()

    for e in range(E_LOCAL):
        slot = e % 2
        pending[0].wait()
        pending[1].wait()
        if e + 1 < E_LOCAL:
            pending = start_load(e + 1, (e + 1) % 2)
        e_global = E_LOCAL * my_x + e
        for c in range(N_CHUNK):
            rows = pl.ds(c * CHUNK, CHUNK)
            xb = xq_ref[rows, :].astype(jnp.float32)
            h = jnp.maximum(
                jnp.dot(xb, w1_buf[slot], preferred_element_type=jnp.float32),
                0.0,
            )
            o = jnp.dot(h, w2_buf[slot], preferred_element_type=jnp.float32)
            mask = aq_ref[rows, :] == e_global
            if e == 0:
                part_ref[rows, :] = jnp.where(mask, o, 0.0).astype(jnp.bfloat16)
            else:
                part_ref[rows, :] = jnp.where(
                    mask, o.astype(jnp.bfloat16), part_ref[rows, :]
                )


def _combine_body(
    part_ref, out_ref, recv_a, q_ref, qh_ref, recv_b2, send_sems, recv_sems,
):
    my_x, my_y, my_z = _idx()
    x_peer = (1 - my_x, my_y, my_z)
    y_peer = (my_x, 1 - my_y, my_z)
    z_peer = (my_x, my_y, 1 - my_z)
    i_hold = my_y == my_x

    _barrier([x_peer, y_peer, z_peer])

    def ds(c):
        return pl.ds(c * CHUNK_C, CHUNK_C)

    rd_a = [
        _remote(part_ref.at[ds(c), :], recv_a.at[ds(c), :],
                send_sems.at[0, c], recv_sems.at[0, c], x_peer)
        for c in range(N_CHUNK_C)
    ]
    rd_b1 = [
        _remote(q_ref.at[ds(c), :], qh_ref.at[ds(c), :],
                send_sems.at[1, c], recv_sems.at[1, c], y_peer)
        for c in range(N_CHUNK_C)
    ]
    rd_b2h = [
        _remote(q_ref.at[ds(c), :], recv_b2.at[ds(c), :],
                send_sems.at[2, c], recv_sems.at[2, c], z_peer)
        for c in range(N_CHUNK_C)
    ]
    rd_b2r = [
        _remote(qh_ref.at[ds(c), :], recv_b2.at[ds(c), :],
                send_sems.at[2, c], recv_sems.at[2, c], z_peer)
        for c in range(N_CHUNK_C)
    ]

    @pl.when(jnp.logical_not(i_hold))
    def _send_partials():
        for c in range(N_CHUNK_C):
            rd_a[c].start()

    for c in range(N_CHUNK_C):
        @pl.when(i_hold)
        def _sum_fwd():
            rd_a[c].wait_recv()
            q_ref[ds(c), :] = (
                part_ref[ds(c), :].astype(jnp.float32)
                + recv_a[ds(c), :].astype(jnp.float32)
            ).astype(jnp.bfloat16)
            rd_b1[c].start()
            rd_b2h[c].start()
            out_ref[pl.ds(my_z * T_Q + c * CHUNK_C, CHUNK_C), :] = q_ref[
                ds(c), :
            ].astype(jnp.float32)

    for c in range(N_CHUNK_C):
        @pl.when(jnp.logical_not(i_hold))
        def _recv_fwd():
            rd_b1[c].wait_recv()
            rd_b2r[c].start()
            out_ref[pl.ds(my_z * T_Q + c * CHUNK_C, CHUNK_C), :] = qh_ref[
                ds(c), :
            ].astype(jnp.float32)

    for c in range(N_CHUNK_C):
        rd_b2h[c].wait_recv()
        out_ref[pl.ds((1 - my_z) * T_Q + c * CHUNK_C, CHUNK_C), :] = recv_b2[
            ds(c), :
        ].astype(jnp.float32)

    for c in range(N_CHUNK_C):
        @pl.when(i_hold)
        def _done_hold():
            rd_b1[c].wait_send()
            rd_b2h[c].wait_send()

        @pl.when(jnp.logical_not(i_hold))
        def _done_recv():
            rd_a[c].wait_send()
            rd_b2r[c].wait_send()


def kernel(x, assign, W1, W2):
    a2 = assign.reshape(T_HALF, 1)

    part = pl.pallas_call(
        _moe_body,
        out_shape=jax.ShapeDtypeStruct((T_Q, D), jnp.bfloat16),
        in_specs=[
            pl.BlockSpec(memory_space=pltpu.VMEM),
            pl.BlockSpec(memory_space=pltpu.VMEM),
            pl.BlockSpec(memory_space=pl.ANY),
            pl.BlockSpec(memory_space=pl.ANY),
        ],
        out_specs=pl.BlockSpec(memory_space=pltpu.VMEM),
        scratch_shapes=[
            pltpu.VMEM((T_Q, D), jnp.bfloat16),
            pltpu.VMEM((T_Q, 1), jnp.int32),
            pltpu.VMEM((2, D, F), jnp.float32),
            pltpu.VMEM((2, F, D), jnp.float32),
            pltpu.SemaphoreType.DMA((2, 2)),
            pltpu.SemaphoreType.DMA((2,)),
            pltpu.SemaphoreType.DMA((2,)),
        ],
        compiler_params=pltpu.CompilerParams(
            collective_id=0, vmem_limit_bytes=VMEM_LIMIT
        ),
    )(x, a2, W1, W2)

    out = pl.pallas_call(
        _combine_body,
        out_shape=jax.ShapeDtypeStruct((T_HALF, D), jnp.float32),
        in_specs=[pl.BlockSpec(memory_space=pltpu.VMEM)],
        out_specs=pl.BlockSpec(memory_space=pltpu.VMEM),
        scratch_shapes=[
            pltpu.VMEM((T_Q, D), jnp.bfloat16),
            pltpu.VMEM((T_Q, D), jnp.bfloat16),
            pltpu.VMEM((T_Q, D), jnp.bfloat16),
            pltpu.VMEM((T_Q, D), jnp.bfloat16),
            pltpu.SemaphoreType.DMA((3, N_CHUNK_C)),
            pltpu.SemaphoreType.DMA((3, N_CHUNK_C)),
        ],
        compiler_params=pltpu.CompilerParams(
            collective_id=1, vmem_limit_bytes=VMEM_LIMIT
        ),
    )(part)

    return out


# device time: 123439 ns/iter; 3.3221x vs baseline; 1.0350x over previous
import jax
import jax.numpy as jnp
from jax import lax
from jax.experimental import pallas as pl
from jax.experimental.pallas import tpu as pltpu

T_HALF = 2048
T_Q = 1024
D = 1024
F = 2048
HF = F // 2
E_LOCAL = 4
CHUNK = 512
N_CHUNK = T_Q // CHUNK
CHUNK_C = 256
N_CHUNK_C = T_Q // CHUNK_C
N_SLICE = E_LOCAL * 2
VMEM_LIMIT = 63 * 1024 * 1024


def _idx():
    return lax.axis_index("x"), lax.axis_index("y"), lax.axis_index("z")


def _barrier(peers):
    barrier_sem = pltpu.get_barrier_semaphore()
    for p in peers:
        pl.semaphore_signal(
            barrier_sem, inc=1, device_id=p,
            device_id_type=pl.DeviceIdType.MESH,
        )
    pl.semaphore_wait(barrier_sem, len(peers))


def _remote(src, dst, send_sem, recv_sem, peer):
    return pltpu.make_async_remote_copy(
        src_ref=src, dst_ref=dst, send_sem=send_sem, recv_sem=recv_sem,
        device_id=peer, device_id_type=pl.DeviceIdType.MESH,
    )


def _body(
    x_ref, a_ref, w1_hbm, w2_hbm, out_ref,
    xq_ref, aq_ref, w1_buf, w2_buf, oacc_ref, part_ref, qa_ref, qh_ref,
    recv_b2, dma_sems, dsp_send, dsp_recv, send_sems, recv_sems,
):
    my_x, my_y, my_z = _idx()
    x_peer = (1 - my_x, my_y, my_z)
    y_peer = (my_x, 1 - my_y, my_z)
    z_peer = (my_x, my_y, 1 - my_z)
    i_hold = my_y == my_x

    _barrier([x_peer, y_peer, z_peer])

    rd_x = _remote(xq_ref, xq_ref, dsp_send.at[0], dsp_recv.at[0], x_peer)
    rd_a = _remote(aq_ref, aq_ref, dsp_send.at[1], dsp_recv.at[1], x_peer)

    @pl.when(i_hold)
    def _sender():
        rows = pl.ds(my_z * T_Q, T_Q)
        xq_ref[:, :] = x_ref[rows, :].astype(jnp.bfloat16)
        aq_ref[:, :] = a_ref[rows, :]
        rd_x.start()
        rd_a.start()

    def start_load(s, slot):
        e, half = divmod(s, 2)
        cp1 = pltpu.make_async_copy(
            w1_hbm.at[e, :, pl.ds(half * HF, HF)], w1_buf.at[slot],
            dma_sems.at[slot, 0],
        )
        cp2 = pltpu.make_async_copy(
            w2_hbm.at[e, pl.ds(half * HF, HF), :], w2_buf.at[slot],
            dma_sems.at[slot, 1],
        )
        cp1.start()
        cp2.start()
        return cp1, cp2

    pending = start_load(0, 0)

    @pl.when(i_hold)
    def _sender_done():
        rd_x.wait_send()
        rd_a.wait_send()

    @pl.when(jnp.logical_not(i_hold))
    def _receiver():
        rd_x.wait_recv()
        rd_a.wait_recv()

    def dsc(c):
        return pl.ds(c * CHUNK_C, CHUNK_C)

    rd_sa = [
        _remote(part_ref.at[dsc(c), :], qa_ref.at[dsc(c), :],
                send_sems.at[0, c], recv_sems.at[0, c], x_peer)
        for c in range(N_CHUNK_C)
    ]
    rd_b1 = [
        _remote(qa_ref.at[dsc(c), :], qh_ref.at[dsc(c), :],
                send_sems.at[1, c], recv_sems.at[1, c], y_peer)
        for c in range(N_CHUNK_C)
    ]
    rd_b2h = [
        _remote(qa_ref.at[dsc(c), :], recv_b2.at[dsc(c), :],
                send_sems.at[2, c], recv_sems.at[2, c], z_peer)
        for c in range(N_CHUNK_C)
    ]
    rd_b2r = [
        _remote(qh_ref.at[dsc(c), :], recv_b2.at[dsc(c), :],
                send_sems.at[2, c], recv_sems.at[2, c], z_peer)
        for c in range(N_CHUNK_C)
    ]

    def gemm_chunk(c, e, half, slot):
        rows = pl.ds(c * CHUNK, CHUNK)
        xb = xq_ref[rows, :].astype(jnp.float32)
        hh = jnp.maximum(
            jnp.dot(xb, w1_buf[slot], preferred_element_type=jnp.float32),
            0.0,
        )
        ob = jnp.dot(hh, w2_buf[slot], preferred_element_type=jnp.float32)
        if half == 0:
            oacc_ref[rows, :] = ob
        else:
            o = oacc_ref[rows, :] + ob
            mask = aq_ref[rows, :] == E_LOCAL * my_x + e
            if e == 0:
                part_ref[rows, :] = jnp.where(mask, o, 0.0).astype(jnp.bfloat16)
            else:
                part_ref[rows, :] = jnp.where(
                    mask, o.astype(jnp.bfloat16), part_ref[rows, :]
                )

    for s in range(N_SLICE):
        e, half = divmod(s, 2)
        slot = s % 2
        pending[0].wait()
        pending[1].wait()
        if s + 1 < N_SLICE:
            pending = start_load(s + 1, 1 - slot)
        if e < E_LOCAL - 1:
            lax.fori_loop(
                0, N_CHUNK,
                lambda c, _, e=e, half=half, slot=slot: (
                    gemm_chunk(c, e, half, slot), 0
                )[1],
                0,
            )
        else:
            for c in range(N_CHUNK):
                gemm_chunk(c, e, half, slot)
                if half == 1:
                    @pl.when(jnp.logical_not(i_hold))
                    def _send_partials():
                        rd_sa[2 * c].start()
                        rd_sa[2 * c + 1].start()

    for c in range(N_CHUNK_C):
        @pl.when(i_hold)
        def _sum_fwd():
            rd_sa[c].wait_recv()
            t32 = part_ref[dsc(c), :].astype(jnp.float32) + qa_ref[
                dsc(c), :
            ].astype(jnp.float32)
            qa_ref[dsc(c), :] = t32.astype(jnp.bfloat16)
            rd_b1[c].start()
            rd_b2h[c].start()
            out_ref[pl.ds(my_z * T_Q + c * CHUNK_C, CHUNK_C), :] = t32

    for c in range(N_CHUNK_C):
        @pl.when(jnp.logical_not(i_hold))
        def _recv_fwd():
            rd_b1[c].wait_recv()
            rd_b2r[c].start()
            out_ref[pl.ds(my_z * T_Q + c * CHUNK_C, CHUNK_C), :] = qh_ref[
                dsc(c), :
            ].astype(jnp.float32)

    for c in range(N_CHUNK_C):
        rd_b2h[c].wait_recv()
        out_ref[pl.ds((1 - my_z) * T_Q + c * CHUNK_C, CHUNK_C), :] = recv_b2[
            dsc(c), :
        ].astype(jnp.float32)

    for c in range(N_CHUNK_C):
        @pl.when(i_hold)
        def _done_hold():
            rd_b1[c].wait_send()
            rd_b2h[c].wait_send()

        @pl.when(jnp.logical_not(i_hold))
        def _done_recv():
            rd_sa[c].wait_send()
            rd_b2r[c].wait_send()


def kernel(x, assign, W1, W2):
    a2 = assign.reshape(T_HALF, 1)

    out = pl.pallas_call(
        _body,
        out_shape=jax.ShapeDtypeStruct((T_HALF, D), jnp.float32),
        in_specs=[
            pl.BlockSpec(memory_space=pltpu.VMEM),
            pl.BlockSpec(memory_space=pltpu.VMEM),
            pl.BlockSpec(memory_space=pl.ANY),
            pl.BlockSpec(memory_space=pl.ANY),
        ],
        out_specs=pl.BlockSpec(memory_space=pltpu.VMEM),
        scratch_shapes=[
            pltpu.VMEM((T_Q, D), jnp.bfloat16),
            pltpu.VMEM((T_Q, 1), jnp.int32),
            pltpu.VMEM((2, D, HF), jnp.float32),
            pltpu.VMEM((2, HF, D), jnp.float32),
            pltpu.VMEM((T_Q, D), jnp.float32),
            pltpu.VMEM((T_Q, D), jnp.bfloat16),
            pltpu.VMEM((T_Q, D), jnp.bfloat16),
            pltpu.VMEM((T_Q, D), jnp.bfloat16),
            pltpu.VMEM((T_Q, D), jnp.bfloat16),
            pltpu.SemaphoreType.DMA((2, 2)),
            pltpu.SemaphoreType.DMA((2,)),
            pltpu.SemaphoreType.DMA((2,)),
            pltpu.SemaphoreType.DMA((3, N_CHUNK_C)),
            pltpu.SemaphoreType.DMA((3, N_CHUNK_C)),
        ],
        compiler_params=pltpu.CompilerParams(
            collective_id=0, vmem_limit_bytes=VMEM_LIMIT
        ),
    )(x, a2, W1, W2)

    return out
